# Initial kernel scaffold; baseline (speedup 1.0000x reference)
import functools

import jax
import jax.numpy as jnp
from jax import lax
from jax.experimental import pallas as pl
from jax.experimental.pallas import tpu as pltpu
from jax.experimental.pallas import tpu_sc as plsc

CROP = 7
NC, NS = 2, 16
NW = NC * NS
RPW = 32
NPAD = NW * RPW
SPAN = 11
LANE = 16


def _roi_pool_sc(data_t, params, C, H, W):
    nchunk = C // LANE
    mesh = plsc.VectorSubcoreMesh(
        core_axis_name="c", subcore_axis_name="s",
        num_cores=NC, num_subcores=NS)

    @functools.partial(
        pl.kernel,
        out_type=jax.ShapeDtypeStruct((NPAD, CROP, CROP, C), jnp.float32),
        mesh=mesh,
        scratch_types=[
            pltpu.VMEM((RPW, 32), jnp.int32),
            pltpu.VMEM((SPAN, W, C), jnp.float32),
            pltpu.VMEM((W, C), jnp.float32),
            pltpu.VMEM((CROP, CROP, C), jnp.float32),
            pltpu.SemaphoreType.DMA,
        ],
    )
    def body(data_hbm, params_hbm, out_hbm, params_v, slots_v, m1_v, out_v,
             sem):
        wid = lax.axis_index("c") * NS + lax.axis_index("s")
        pltpu.sync_copy(params_hbm.at[pl.ds(wid * RPW, RPW), :], params_v)
        minus_inf = jnp.full((LANE,), -jnp.inf, jnp.float32)
        zeros = jnp.zeros((LANE,), jnp.float32)

        def roi_body(i, _):
            r = wid * RPW + i
            b = params_v[i, 0]
            ws0 = params_v[i, 15]
            we6 = params_v[i, 28]

            def ph_body(ph, _):
                hs = params_v[i, 1 + ph]
                he = params_v[i, 8 + ph]
                span = he - hs

                def fire(j, _):
                    pltpu.async_copy(data_hbm.at[b, hs + j], slots_v.at[j],
                                     sem)
                    return 0
                lax.fori_loop(0, span, fire, 0)

                def drain(j, _):
                    pltpu.make_async_copy(data_hbm.at[b, 0], slots_v.at[0],
                                          sem).wait()
                    return 0
                lax.fori_loop(0, span, drain, 0)

                def w_body(w, _):
                    def j_body(j, accs):
                        return tuple(
                            jnp.maximum(accs[k],
                                        slots_v[j, w, pl.ds(k * LANE, LANE)])
                            for k in range(nchunk))
                    accs = lax.fori_loop(0, span, j_body,
                                         (minus_inf,) * nchunk)
                    for k in range(nchunk):
                        m1_v[w, pl.ds(k * LANE, LANE)] = accs[k]
                    return 0
                lax.fori_loop(ws0, we6, w_body, 0)

                def pw_body(pw, _):
                    ws = params_v[i, 15 + pw]
                    we = params_v[i, 22 + pw]

                    def wb(w, accs):
                        return tuple(
                            jnp.maximum(accs[k],
                                        m1_v[w, pl.ds(k * LANE, LANE)])
                            for k in range(nchunk))
                    accs = lax.fori_loop(ws, we, wb, (minus_inf,) * nchunk)
                    for k in range(nchunk):
                        out_v[ph, pw, pl.ds(k * LANE, LANE)] = jnp.where(
                            accs[k] < -1e30, zeros, accs[k])
                    return 0
                lax.fori_loop(0, CROP, pw_body, 0)
                return 0

            lax.fori_loop(0, CROP, ph_body, 0)
            pltpu.sync_copy(out_v, out_hbm.at[r])
            return 0

        lax.fori_loop(0, RPW, roi_body, 0)

    return body(data_t, params)


def kernel(data, rois, roibatches, spatial_scale):
    B, C, H, W = data.shape
    N = rois.shape[0]
    scale = jnp.asarray(spatial_scale, jnp.float32)

    data_t = jnp.transpose(data, (0, 2, 3, 1))

    sw = jnp.round(rois[:, 0] * scale).astype(jnp.int32)
    sh = jnp.round(rois[:, 1] * scale).astype(jnp.int32)
    ew = jnp.round(rois[:, 2] * scale).astype(jnp.int32)
    eh = jnp.round(rois[:, 3] * scale).astype(jnp.int32)
    roi_w = jnp.maximum(ew - sw + 1, 1).astype(jnp.float32)
    roi_h = jnp.maximum(eh - sh + 1, 1).astype(jnp.float32)
    bin_h = roi_h / CROP
    bin_w = roi_w / CROP
    p = jnp.arange(CROP, dtype=jnp.float32)
    hstart = jnp.clip(
        jnp.floor(p[None, :] * bin_h[:, None]).astype(jnp.int32)
        + sh[:, None], 0, H)
    hend = jnp.clip(
        jnp.ceil((p[None, :] + 1.0) * bin_h[:, None]).astype(jnp.int32)
        + sh[:, None], 0, H)
    wstart = jnp.clip(
        jnp.floor(p[None, :] * bin_w[:, None]).astype(jnp.int32)
        + sw[:, None], 0, W)
    wend = jnp.clip(
        jnp.ceil((p[None, :] + 1.0) * bin_w[:, None]).astype(jnp.int32)
        + sw[:, None], 0, W)

    params = jnp.concatenate(
        [roibatches.astype(jnp.int32)[:, None], hstart, hend, wstart, wend,
         jnp.zeros((N, 3), jnp.int32)], axis=1)
    params = jnp.pad(params, ((0, NPAD - N), (0, 0)))

    out = _roi_pool_sc(data_t, params, C, H, W)
    return jnp.transpose(out[:N], (0, 3, 1, 2))


# speedup vs baseline: 17.7895x; 17.7895x over previous
import functools

import jax
import jax.numpy as jnp
from jax import lax
from jax.experimental import pallas as pl
from jax.experimental.pallas import tpu as pltpu
from jax.experimental.pallas import tpu_sc as plsc

CROP = 7
NC, NS = 2, 16
NW = NC * NS
RPW = 32
NPAD = NW * RPW
SPAN = 11
LANE = 16


def _roi_pool_sc(data_t, params, C, H, W):
    nchunk = C // LANE
    mesh = plsc.VectorSubcoreMesh(
        core_axis_name="c", subcore_axis_name="s",
        num_cores=NC, num_subcores=NS)

    @functools.partial(
        pl.kernel,
        out_type=jax.ShapeDtypeStruct((NPAD, CROP, CROP, C), jnp.float32),
        mesh=mesh,
        scratch_types=[
            pltpu.VMEM((RPW, 32), jnp.int32),
            pltpu.VMEM((SPAN, W, C), jnp.float32),
            pltpu.VMEM((W, C), jnp.float32),
            pltpu.VMEM((CROP, CROP, C), jnp.float32),
            pltpu.SemaphoreType.DMA,
        ],
    )
    def body(data_hbm, params_hbm, out_hbm, params_v, slots_v, m1_v, out_v,
             sem):
        wid = lax.axis_index("c") * NS + lax.axis_index("s")
        pltpu.sync_copy(params_hbm.at[pl.ds(wid * RPW, RPW), :], params_v)
        minus_inf = jnp.full((LANE,), -jnp.inf, jnp.float32)
        zeros = jnp.zeros((LANE,), jnp.float32)

        def roi_body(i, _):
            r = wid * RPW + i
            r0 = params_v[i, pl.ds(0, LANE)]
            r1 = params_v[i, pl.ds(LANE, LANE)]
            b = r0[0]
            ws0 = r1[0]
            we6 = r1[13]

            for ph in range(CROP):
                hs = r0[1 + ph]
                he = r0[8 + ph]
                span = he - hs

                def fire(j, _):
                    pltpu.async_copy(data_hbm.at[b, hs + j], slots_v.at[j],
                                     sem)
                    return 0
                lax.fori_loop(0, span, fire, 0)

                def drain(j, _):
                    pltpu.make_async_copy(data_hbm.at[b, 0], slots_v.at[0],
                                          sem).wait()
                    return 0
                lax.fori_loop(0, span, drain, 0)

                def w_body(w, _):
                    def j_body(j, accs):
                        return tuple(
                            jnp.maximum(accs[k],
                                        slots_v[j, w, pl.ds(k * LANE, LANE)])
                            for k in range(nchunk))
                    accs = lax.fori_loop(0, span, j_body,
                                         (minus_inf,) * nchunk)
                    for k in range(nchunk):
                        m1_v[w, pl.ds(k * LANE, LANE)] = accs[k]
                    return 0
                lax.fori_loop(ws0, we6, w_body, 0)

                for pw in range(CROP):
                    ws = r1[pw]
                    we = r1[7 + pw]

                    def wb(w, accs):
                        return tuple(
                            jnp.maximum(accs[k],
                                        m1_v[w, pl.ds(k * LANE, LANE)])
                            for k in range(nchunk))
                    accs = lax.fori_loop(ws, we, wb, (minus_inf,) * nchunk)
                    for k in range(nchunk):
                        out_v[ph, pw, pl.ds(k * LANE, LANE)] = jnp.where(
                            accs[k] < -1e30, zeros, accs[k])

            pltpu.sync_copy(out_v, out_hbm.at[r])
            return 0

        lax.fori_loop(0, RPW, roi_body, 0)

    return body(data_t, params)


def kernel(data, rois, roibatches, spatial_scale):
    B, C, H, W = data.shape
    N = rois.shape[0]
    scale = jnp.asarray(spatial_scale, jnp.float32)

    data_t = jnp.transpose(data, (0, 2, 3, 1))

    sw = jnp.round(rois[:, 0] * scale).astype(jnp.int32)
    sh = jnp.round(rois[:, 1] * scale).astype(jnp.int32)
    ew = jnp.round(rois[:, 2] * scale).astype(jnp.int32)
    eh = jnp.round(rois[:, 3] * scale).astype(jnp.int32)
    roi_w = jnp.maximum(ew - sw + 1, 1).astype(jnp.float32)
    roi_h = jnp.maximum(eh - sh + 1, 1).astype(jnp.float32)
    bin_h = roi_h / CROP
    bin_w = roi_w / CROP
    p = jnp.arange(CROP, dtype=jnp.float32)
    hstart = jnp.clip(
        jnp.floor(p[None, :] * bin_h[:, None]).astype(jnp.int32)
        + sh[:, None], 0, H)
    hend = jnp.clip(
        jnp.ceil((p[None, :] + 1.0) * bin_h[:, None]).astype(jnp.int32)
        + sh[:, None], 0, H)
    wstart = jnp.clip(
        jnp.floor(p[None, :] * bin_w[:, None]).astype(jnp.int32)
        + sw[:, None], 0, W)
    wend = jnp.clip(
        jnp.ceil((p[None, :] + 1.0) * bin_w[:, None]).astype(jnp.int32)
        + sw[:, None], 0, W)

    pad1 = jnp.zeros((N, 1), jnp.int32)
    params = jnp.concatenate(
        [roibatches.astype(jnp.int32)[:, None], hstart, hend, pad1,
         wstart, wend, jnp.zeros((N, 2), jnp.int32)], axis=1)
    params = jnp.pad(params, ((0, NPAD - N), (0, 0)))

    out = _roi_pool_sc(data_t, params, C, H, W)
    return jnp.transpose(out[:N], (0, 3, 1, 2))


# speedup vs baseline: 34.6501x; 1.9478x over previous
import functools

import jax
import jax.numpy as jnp
from jax import lax
from jax.experimental import pallas as pl
from jax.experimental.pallas import tpu as pltpu
from jax.experimental.pallas import tpu_sc as plsc

CROP = 7
NC, NS = 2, 16
NW = NC * NS
RPW = 32
NPAD = NW * RPW
UPT = RPW * CROP
LANE = 16
NLEG = 4


def _pyramid_tc(data_t, B, H, W, C):
    def body(x_ref, out_ref, scratch):
        lvl = pl.program_id(1)

        @pl.when(lvl == 0)
        def _():
            scratch[...] = x_ref[0]

        for k in (1, 2, 3):
            @pl.when(lvl == k)
            def _():
                d = 1 << (k - 1)
                cur = scratch[...]
                shifted = jnp.concatenate(
                    [cur[d:], jnp.broadcast_to(cur[H - 1:], (d, W, C))],
                    axis=0)
                scratch[...] = jnp.maximum(cur, shifted)

        out_ref[0, 0] = scratch[...]

    return pl.pallas_call(
        body,
        grid=(B, 4),
        in_specs=[pl.BlockSpec((1, H, W, C), lambda b, l: (b, 0, 0, 0))],
        out_specs=pl.BlockSpec((1, 1, H, W, C),
                               lambda b, l: (l, b, 0, 0, 0)),
        out_shape=jax.ShapeDtypeStruct((4, B, H, W, C), jnp.float32),
        scratch_shapes=[pltpu.VMEM((H, W, C), jnp.float32)],
    )(data_t)


def _roi_pool_sc(ph_flat, params_u, C, W):
    nchunk = C // LANE
    mesh = plsc.VectorSubcoreMesh(
        core_axis_name="c", subcore_axis_name="s",
        num_cores=NC, num_subcores=NS)

    @functools.partial(
        pl.kernel,
        out_type=jax.ShapeDtypeStruct((NPAD, CROP, CROP, C), jnp.float32),
        mesh=mesh,
        scratch_types=[
            pltpu.VMEM((UPT, LANE), jnp.int32),
            pltpu.VMEM((2 * NLEG, W, C), jnp.float32),
            pltpu.VMEM((CROP, CROP, C), jnp.float32),
            pltpu.SemaphoreType.DMA,
            pltpu.SemaphoreType.DMA,
            pltpu.SemaphoreType.DMA,
            pltpu.SemaphoreType.DMA,
        ],
    )
    def body(ph_hbm, pu_hbm, out_hbm, pu_v, slots_v, out_v,
             sem0, sem1, sem2, sem3):
        sems = (sem0, sem1, sem2, sem3)
        wid = lax.axis_index("c") * NS + lax.axis_index("s")
        pltpu.sync_copy(pu_hbm.at[pl.ds(wid * UPT, UPT), :], pu_v)
        minus_inf = jnp.full((LANE,), -jnp.inf, jnp.float32)
        zeros = jnp.zeros((LANE,), jnp.float32)

        def fire(u, leg):
            pv = pu_v[u, pl.ds(0, LANE)]
            pltpu.async_copy(ph_hbm.at[pv[0]], slots_v.at[2 * leg],
                             sems[leg])
            pltpu.async_copy(ph_hbm.at[pv[1]], slots_v.at[2 * leg + 1],
                             sems[leg])

        for leg in range(NLEG):
            fire(leg, leg)

        def group(g, _):
            for leg in range(NLEG):
                u = g * NLEG + leg
                for _ in range(2):
                    pltpu.make_async_copy(ph_hbm.at[0], slots_v.at[0],
                                          sems[leg]).wait()

                pv = pu_v[u, pl.ds(0, LANE)]
                ph = u % CROP
                for pw in range(CROP):
                    ws = pv[2 + pw]
                    we = pv[9 + pw]

                    def wb(w, accs):
                        return tuple(
                            jnp.maximum(
                                jnp.maximum(
                                    accs[k],
                                    slots_v[2 * leg, w,
                                            pl.ds(k * LANE, LANE)]),
                                slots_v[2 * leg + 1, w,
                                        pl.ds(k * LANE, LANE)])
                            for k in range(nchunk))
                    accs = lax.fori_loop(ws, we, wb, (minus_inf,) * nchunk)
                    for k in range(nchunk):
                        out_v[ph, pw, pl.ds(k * LANE, LANE)] = jnp.where(
                            accs[k] < -1e30, zeros, accs[k])

                @pl.when(ph == CROP - 1)
                def _():
                    pltpu.sync_copy(out_v, out_hbm.at[wid * RPW + u // CROP])

                @pl.when(u + NLEG < UPT)
                def _():
                    fire(u + NLEG, leg)
            return 0

        lax.fori_loop(0, UPT // NLEG, group, 0)

    return body(ph_flat, params_u)


def kernel(data, rois, roibatches, spatial_scale):
    B, C, H, W = data.shape
    N = rois.shape[0]
    scale = jnp.asarray(spatial_scale, jnp.float32)

    data_t = jnp.transpose(data, (0, 2, 3, 1))
    ph_pyr = _pyramid_tc(data_t, B, H, W, C)
    ph_flat = ph_pyr.reshape(4 * B * H, W, C)

    sw = jnp.round(rois[:, 0] * scale).astype(jnp.int32)
    sh = jnp.round(rois[:, 1] * scale).astype(jnp.int32)
    ew = jnp.round(rois[:, 2] * scale).astype(jnp.int32)
    eh = jnp.round(rois[:, 3] * scale).astype(jnp.int32)
    roi_w = jnp.maximum(ew - sw + 1, 1).astype(jnp.float32)
    roi_h = jnp.maximum(eh - sh + 1, 1).astype(jnp.float32)
    bin_h = roi_h / CROP
    bin_w = roi_w / CROP
    p = jnp.arange(CROP, dtype=jnp.float32)
    hstart = jnp.clip(
        jnp.floor(p[None, :] * bin_h[:, None]).astype(jnp.int32)
        + sh[:, None], 0, H)
    hend = jnp.clip(
        jnp.ceil((p[None, :] + 1.0) * bin_h[:, None]).astype(jnp.int32)
        + sh[:, None], 0, H)
    wstart = jnp.clip(
        jnp.floor(p[None, :] * bin_w[:, None]).astype(jnp.int32)
        + sw[:, None], 0, W)
    wend = jnp.clip(
        jnp.ceil((p[None, :] + 1.0) * bin_w[:, None]).astype(jnp.int32)
        + sw[:, None], 0, W)

    span_h = hend - hstart
    lvl = ((span_h >= 2).astype(jnp.int32)
           + (span_h >= 4).astype(jnp.int32)
           + (span_h >= 8).astype(jnp.int32))
    pow2 = jnp.left_shift(jnp.int32(1), lvl)
    b_ = roibatches.astype(jnp.int32)[:, None]
    ra = (lvl * B + b_) * H + hstart
    rb = (lvl * B + b_) * H + (hend - pow2)
    emptyh = span_h <= 0
    ra = jnp.where(emptyh, 0, ra)
    rb = jnp.where(emptyh, 0, rb)
    ws_u = jnp.broadcast_to(wstart[:, None, :], (N, CROP, CROP))
    we_u = jnp.where(emptyh[:, :, None], wstart[:, None, :],
                     wend[:, None, :])
    params_u = jnp.concatenate(
        [ra[:, :, None], rb[:, :, None], ws_u, we_u], axis=2)
    params_u = jnp.pad(params_u, ((0, NPAD - N), (0, 0), (0, 0)))
    params_u = params_u.reshape(NPAD * CROP, LANE)

    out = _roi_pool_sc(ph_flat, params_u, C, W)
    return jnp.transpose(out[:N], (0, 3, 1, 2))
